# Initial kernel scaffold; baseline (speedup 1.0000x reference)
import jax
import jax.numpy as jnp
from jax.experimental import pallas as pl


def kernel(x, edge_index, edge_attr, batch, params):
    raise NotImplementedError("write your pallas kernel here")


# speedup vs baseline: 1.4304x; 1.4304x over previous
import functools

import jax
import jax.numpy as jnp
from jax import lax
from jax.experimental import pallas as pl
from jax.experimental.pallas import tpu as pltpu
from jax.experimental.pallas import tpu_sc as plsc

N = 10000
E = 160000
C = 16
HID = 64
H2 = C * C
LAYER_DIM = 154
N_LAYERS = 6
N_GRAPHS = 64

NC = 2
NS = 16
NW = NC * NS
CH = 128
K = 40
EPW = K * CH
EP = NW * EPW
RPT = 640
NACC = NS * RPT

TE = 2048
TN = 2000

_DP = dict(preferred_element_type=jnp.float32, precision=lax.Precision.DEFAULT)
_XP = dict(preferred_element_type=jnp.float32, precision=lax.Precision.HIGHEST)


@functools.cache
def _sc_kernels():
    mesh = plsc.VectorSubcoreMesh(core_axis_name="c", subcore_axis_name="s")

    @functools.partial(
        pl.kernel,
        out_type=jax.ShapeDtypeStruct((EP, C), jnp.float32),
        mesh=mesh,
        compiler_params=pltpu.CompilerParams(use_tc_tiling_on_sc=False),
        scratch_types=[
            pltpu.VMEM((K, CH), jnp.int32),
            pltpu.VMEM((CH, C), jnp.float32),
            pltpu.VMEM((CH, C), jnp.float32),
            pltpu.SemaphoreType.DMA,
            pltpu.SemaphoreType.DMA,
        ],
    )
    def sc_gather(table_hbm, idx_hbm, out_hbm, idx_v, rows0, rows1, sem0, sem1):
        cid = lax.axis_index("c")
        sid = lax.axis_index("s")
        wid = sid * NC + cid
        pltpu.sync_copy(idx_hbm.at[wid], idx_v)
        base = wid * EPW

        def body(t, carry):
            j0 = 2 * t
            j1 = j0 + 1
            c0 = pltpu.async_copy(table_hbm.at[idx_v.at[j0]], rows0, sem0)
            c1 = pltpu.async_copy(table_hbm.at[idx_v.at[j1]], rows1, sem1)
            c0.wait()
            pltpu.sync_copy(rows0, out_hbm.at[pl.ds(base + j0 * CH, CH)])
            c1.wait()
            pltpu.sync_copy(rows1, out_hbm.at[pl.ds(base + j1 * CH, CH)])
            return carry

        lax.fori_loop(0, K // 2, body, 0)

    @functools.partial(
        pl.kernel,
        out_type=jax.ShapeDtypeStruct((NC * NACC, C), jnp.float32),
        mesh=mesh,
        compiler_params=pltpu.CompilerParams(use_tc_tiling_on_sc=False),
        scratch_types=[
            pltpu.VMEM((K, CH), jnp.int32),
            pltpu.VMEM((CH, C), jnp.float32),
            pltpu.VMEM((CH, C), jnp.float32),
            pltpu.VMEM_SHARED((NACC, C), jnp.float32),
            pltpu.SemaphoreType.DMA,
            pltpu.SemaphoreType.DMA,
        ],
    )
    def sc_scatter(msg_hbm, dst_hbm, zeros_hbm, out_hbm,
                   idx_v, chunk0, chunk1, acc_sh, sem0, sem1):
        cid = lax.axis_index("c")
        sid = lax.axis_index("s")
        wid = sid * NC + cid
        pltpu.sync_copy(zeros_hbm, acc_sh.at[pl.ds(sid * RPT, RPT)])
        pltpu.sync_copy(dst_hbm.at[wid], idx_v)
        plsc.subcore_barrier()
        base = wid * EPW

        def body(t, carry):
            j0 = 2 * t
            j1 = j0 + 1
            c0 = pltpu.async_copy(msg_hbm.at[pl.ds(base + j0 * CH, CH)], chunk0, sem0)
            c1 = pltpu.async_copy(msg_hbm.at[pl.ds(base + j1 * CH, CH)], chunk1, sem1)
            c0.wait()
            pltpu.sync_copy(chunk0, acc_sh.at[idx_v.at[j0]], add=True)
            c1.wait()
            pltpu.sync_copy(chunk1, acc_sh.at[idx_v.at[j1]], add=True)
            return carry

        lax.fori_loop(0, K // 2, body, 0)
        plsc.subcore_barrier()
        pltpu.sync_copy(
            acc_sh.at[pl.ds(sid * RPT, RPT)],
            out_hbm.at[pl.ds(cid * NACC + sid * RPT, RPT)],
        )

    return sc_gather, sc_scatter


def _msg_body(ea_ref, xs_ref, w1_ref, b1_ref, w2_ref, b2_ref, o_ref):
    h1 = jnp.maximum(jnp.dot(ea_ref[...], w1_ref[...], **_DP) + b1_ref[...], 0.0)
    hw = jnp.dot(h1, w2_ref[...], **_DP) + b2_ref[...]
    hwb = hw.astype(jnp.bfloat16).astype(jnp.float32)
    xs = xs_ref[...].astype(jnp.bfloat16).astype(jnp.float32)
    acc = xs[:, 0:1] * hwb[:, 0:C]
    for i in range(1, C):
        acc = acc + xs[:, i:i + 1] * hwb[:, C * i:C * (i + 1)]
    o_ref[...] = acc


_msg_call = pl.pallas_call(
    _msg_body,
    grid=(EP // TE,),
    in_specs=[
        pl.BlockSpec((TE, C), lambda j: (j, 0)),
        pl.BlockSpec((TE, C), lambda j: (j, 0)),
        pl.BlockSpec((C, HID), lambda j: (0, 0)),
        pl.BlockSpec((1, HID), lambda j: (0, 0)),
        pl.BlockSpec((HID, H2), lambda j: (0, 0)),
        pl.BlockSpec((1, H2), lambda j: (0, 0)),
    ],
    out_specs=pl.BlockSpec((TE, C), lambda j: (j, 0)),
    out_shape=jax.ShapeDtypeStruct((EP, C), jnp.float32),
)


def _combine_body(p0_ref, p1_ref, x_ref, root_ref, bias_ref, o_ref):
    o_ref[...] = (p0_ref[...] + p1_ref[...]
                  + jnp.dot(x_ref[...], root_ref[...], **_DP) + bias_ref[...])


_combine_call = pl.pallas_call(
    _combine_body,
    grid=(N // TN,),
    in_specs=[
        pl.BlockSpec((TN, C), lambda j: (j, 0)),
        pl.BlockSpec((TN, C), lambda j: (j, 0)),
        pl.BlockSpec((TN, C), lambda j: (j, 0)),
        pl.BlockSpec((C, C), lambda j: (0, 0)),
        pl.BlockSpec((1, C), lambda j: (0, 0)),
    ],
    out_specs=pl.BlockSpec((TN, C), lambda j: (j, 0)),
    out_shape=jax.ShapeDtypeStruct((N, C), jnp.float32),
)


def _pool_body(p0_ref, p1_ref, x_ref, root_ref, bias_ref, batch_ref, o_ref):
    j = pl.program_id(0)
    xx = (p0_ref[...] + p1_ref[...]
          + jnp.dot(x_ref[...], root_ref[...], **_DP) + bias_ref[...])
    b = batch_ref[0]
    g = lax.broadcasted_iota(jnp.int32, (N_GRAPHS, TN), 0)
    oh = jnp.where(g == b, 1.0, 0.0).astype(jnp.float32)
    part = jnp.dot(oh, xx, **_XP)

    @pl.when(j == 0)
    def _():
        o_ref[...] = part

    @pl.when(j != 0)
    def _():
        o_ref[...] = o_ref[...] + part


_pool_call = pl.pallas_call(
    _pool_body,
    grid=(N // TN,),
    in_specs=[
        pl.BlockSpec((TN, C), lambda j: (j, 0)),
        pl.BlockSpec((TN, C), lambda j: (j, 0)),
        pl.BlockSpec((TN, C), lambda j: (j, 0)),
        pl.BlockSpec((C, C), lambda j: (0, 0)),
        pl.BlockSpec((1, C), lambda j: (0, 0)),
        pl.BlockSpec((1, 1, TN), lambda j: (j, 0, 0)),
    ],
    out_specs=pl.BlockSpec((N_GRAPHS, C), lambda j: (0, 0)),
    out_shape=jax.ShapeDtypeStruct((N_GRAPHS, C), jnp.float32),
)


def _head_body(pooled_ref, lw_ref, lb_ref, w1s_ref, b1s_ref, w2s_ref, b2s_ref,
               sw_ref, sb_ref, rw_ref, rb_ref, o_size_ref, o_rse_ref):
    out = jnp.dot(pooled_ref[...], lw_ref[...], **_DP) + lb_ref[...]
    for li in range(N_LAYERS):
        h = jnp.maximum(jnp.dot(out, w1s_ref[li], **_DP) + b1s_ref[li], 0.0)
        h = jnp.dot(h, w2s_ref[li], **_DP) + b2s_ref[li]
        out = out + h
    o_size_ref[...] = jnp.dot(out, sw_ref[...], **_DP) + sb_ref[...]
    o_rse_ref[...] = jnp.dot(out, rw_ref[...], **_DP) + rb_ref[...]


_head_call = pl.pallas_call(
    _head_body,
    out_shape=(
        jax.ShapeDtypeStruct((N_GRAPHS, 7), jnp.float32),
        jax.ShapeDtypeStruct((N_GRAPHS, 1), jnp.float32),
    ),
)


def kernel(x, edge_index, edge_attr, batch, params):
    f32 = jnp.float32
    src = edge_index[0]
    dst = edge_index[1]
    pad = EP - E
    src_p = jnp.concatenate([src, jnp.zeros((pad,), jnp.int32)]).reshape(NW, K, CH)
    sink = N + (jnp.arange(pad, dtype=jnp.int32) % (NACC - N))
    dst_p = jnp.concatenate([dst, sink]).reshape(NW, K, CH)
    ea_p = jnp.concatenate([edge_attr, jnp.zeros((pad, C), f32)], axis=0)
    zeros_tile = jnp.zeros((RPT, C), f32)
    batch3 = batch.reshape(N // TN, 1, TN)

    sc_gather, sc_scatter = _sc_kernels()

    def conv(xin, p):
        xs = sc_gather(xin, src_p)
        msg = _msg_call(ea_p, xs, p["w1"], p["b1"].reshape(1, HID),
                        p["w2"], p["b2"].reshape(1, H2))
        return sc_scatter(msg, dst_p, zeros_tile)

    p1 = params["conv1"]
    parts1 = conv(x, p1)
    x1 = _combine_call(parts1[:NACC], parts1[NACC:], x,
                       p1["root"], p1["bias"].reshape(1, C))

    p2 = params["conv2"]
    parts2 = conv(x1, p2)
    pooled = _pool_call(parts2[:NACC], parts2[NACC:], x1,
                        p2["root"], p2["bias"].reshape(1, C), batch3)

    w1s = jnp.stack([lp["w1"] for lp in params["layers"]])
    b1s = jnp.stack([lp["b1"].reshape(1, LAYER_DIM) for lp in params["layers"]])
    w2s = jnp.stack([lp["w2"] for lp in params["layers"]])
    b2s = jnp.stack([lp["b2"].reshape(1, LAYER_DIM) for lp in params["layers"]])
    size_logits, rse = _head_call(
        pooled, params["lin0"]["w"], params["lin0"]["b"].reshape(1, LAYER_DIM),
        w1s, b1s, w2s, b2s,
        params["size"]["w"], params["size"]["b"].reshape(1, 7),
        params["rse"]["w"], params["rse"]["b"].reshape(1, 1),
    )
    return (size_logits, rse)


# speedup vs baseline: 2.8550x; 1.9960x over previous
import functools

import jax
import jax.numpy as jnp
from jax import lax
from jax.experimental import pallas as pl
from jax.experimental.pallas import tpu as pltpu
from jax.experimental.pallas import tpu_sc as plsc

N = 10000
E = 160000
C = 16
HID = 64
H2 = C * C
LAYER_DIM = 154
N_LAYERS = 6
N_GRAPHS = 64

NC = 2
NS = 16
NW = NC * NS
CH = 128
K = 40
EPW = K * CH
EP = NW * EPW
RPT = 640
NACC = NS * RPT

TE = 2048
TN = 2000

_DP = dict(preferred_element_type=jnp.float32, precision=lax.Precision.DEFAULT)
_XP = dict(preferred_element_type=jnp.float32, precision=lax.Precision.HIGHEST)


@functools.cache
def _sc_kernels():
    mesh = plsc.VectorSubcoreMesh(core_axis_name="c", subcore_axis_name="s")

    @functools.partial(
        pl.kernel,
        out_type=jax.ShapeDtypeStruct((EP, C), jnp.float32),
        mesh=mesh,
        compiler_params=pltpu.CompilerParams(use_tc_tiling_on_sc=False),
        scratch_types=[
            pltpu.VMEM((K, CH), jnp.int32),
            pltpu.VMEM((EPW, C), jnp.float32),
            pltpu.SemaphoreType.DMA,
        ],
    )
    def sc_gather(table_hbm, idx_hbm, out_hbm, idx_v, rows_buf, sem):
        cid = lax.axis_index("c")
        sid = lax.axis_index("s")
        wid = sid * NC + cid
        pltpu.sync_copy(idx_hbm.at[wid], idx_v)
        base = wid * EPW

        def fire(j, carry):
            pltpu.async_copy(table_hbm.at[idx_v.at[j]],
                             rows_buf.at[pl.ds(j * CH, CH)], sem)
            return carry

        lax.fori_loop(0, K, fire, 0)
        pltpu.make_async_copy(out_hbm.at[pl.ds(base, EPW)], rows_buf, sem).wait()
        pltpu.sync_copy(rows_buf, out_hbm.at[pl.ds(base, EPW)])

    @functools.partial(
        pl.kernel,
        out_type=jax.ShapeDtypeStruct((NC * NACC, C), jnp.float32),
        mesh=mesh,
        compiler_params=pltpu.CompilerParams(use_tc_tiling_on_sc=False),
        scratch_types=[
            pltpu.VMEM((K, CH), jnp.int32),
            pltpu.VMEM((EPW, C), jnp.float32),
            pltpu.VMEM_SHARED((NACC, C), jnp.float32),
            pltpu.SemaphoreType.DMA,
            pltpu.SemaphoreType.DMA,
        ],
    )
    def sc_scatter(msg_hbm, dst_hbm, zeros_hbm, out_hbm,
                   idx_v, msg_buf, acc_sh, sem, sem2):
        cid = lax.axis_index("c")
        sid = lax.axis_index("s")
        wid = sid * NC + cid
        pltpu.sync_copy(zeros_hbm, acc_sh.at[pl.ds(sid * RPT, RPT)])
        pltpu.sync_copy(dst_hbm.at[wid], idx_v)
        base = wid * EPW
        pltpu.sync_copy(msg_hbm.at[pl.ds(base, EPW)], msg_buf)
        plsc.subcore_barrier()

        def fire(j, carry):
            pltpu.async_copy(msg_buf.at[pl.ds(j * CH, CH)],
                             acc_sh.at[idx_v.at[j]], sem, add=True)
            return carry

        lax.fori_loop(0, K, fire, 0)
        pltpu.make_async_copy(msg_hbm.at[pl.ds(base, EPW)], msg_buf, sem).wait()
        plsc.subcore_barrier()
        pltpu.sync_copy(
            acc_sh.at[pl.ds(sid * RPT, RPT)],
            out_hbm.at[pl.ds(cid * NACC + sid * RPT, RPT)],
        )

    return sc_gather, sc_scatter


def _msg_body(ea_ref, xs_ref, w1_ref, b1_ref, w2_ref, b2_ref, o_ref):
    h1 = jnp.maximum(jnp.dot(ea_ref[...], w1_ref[...], **_DP) + b1_ref[...], 0.0)
    hw = jnp.dot(h1, w2_ref[...], **_DP) + b2_ref[...]
    hwb = hw.astype(jnp.bfloat16).astype(jnp.float32)
    xs = xs_ref[...].astype(jnp.bfloat16).astype(jnp.float32)
    row = lax.broadcasted_iota(jnp.int32, (C, H2), 0)
    col = lax.broadcasted_iota(jnp.int32, (C, H2), 1)
    sel_r = jnp.where(col // C == row, 1.0, 0.0).astype(jnp.float32)
    xe = jnp.dot(xs, sel_r, **_DP)
    prod = xe * hwb
    p_hi = prod.astype(jnp.bfloat16).astype(jnp.float32)
    p_lo = prod - p_hi
    cc = lax.broadcasted_iota(jnp.int32, (H2, C), 0)
    oo = lax.broadcasted_iota(jnp.int32, (H2, C), 1)
    sel_s = jnp.where(cc % C == oo, 1.0, 0.0).astype(jnp.float32)
    o_ref[...] = (jnp.dot(p_hi, sel_s, **_DP)
                  + jnp.dot(p_lo, sel_s, **_DP))


_msg_call = pl.pallas_call(
    _msg_body,
    grid=(EP // TE,),
    in_specs=[
        pl.BlockSpec((TE, C), lambda j: (j, 0)),
        pl.BlockSpec((TE, C), lambda j: (j, 0)),
        pl.BlockSpec((C, HID), lambda j: (0, 0)),
        pl.BlockSpec((1, HID), lambda j: (0, 0)),
        pl.BlockSpec((HID, H2), lambda j: (0, 0)),
        pl.BlockSpec((1, H2), lambda j: (0, 0)),
    ],
    out_specs=pl.BlockSpec((TE, C), lambda j: (j, 0)),
    out_shape=jax.ShapeDtypeStruct((EP, C), jnp.float32),
)


def _combine_body(p0_ref, p1_ref, x_ref, root_ref, bias_ref, o_ref):
    o_ref[...] = (p0_ref[...] + p1_ref[...]
                  + jnp.dot(x_ref[...], root_ref[...], **_DP) + bias_ref[...])


_combine_call = pl.pallas_call(
    _combine_body,
    grid=(N // TN,),
    in_specs=[
        pl.BlockSpec((TN, C), lambda j: (j, 0)),
        pl.BlockSpec((TN, C), lambda j: (j, 0)),
        pl.BlockSpec((TN, C), lambda j: (j, 0)),
        pl.BlockSpec((C, C), lambda j: (0, 0)),
        pl.BlockSpec((1, C), lambda j: (0, 0)),
    ],
    out_specs=pl.BlockSpec((TN, C), lambda j: (j, 0)),
    out_shape=jax.ShapeDtypeStruct((N, C), jnp.float32),
)


def _pool_body(p0_ref, p1_ref, x_ref, root_ref, bias_ref, batch_ref, o_ref):
    j = pl.program_id(0)
    xx = (p0_ref[...] + p1_ref[...]
          + jnp.dot(x_ref[...], root_ref[...], **_DP) + bias_ref[...])
    b = batch_ref[0]
    g = lax.broadcasted_iota(jnp.int32, (N_GRAPHS, TN), 0)
    oh = jnp.where(g == b, 1.0, 0.0).astype(jnp.float32)
    part = jnp.dot(oh, xx, **_XP)

    @pl.when(j == 0)
    def _():
        o_ref[...] = part

    @pl.when(j != 0)
    def _():
        o_ref[...] = o_ref[...] + part


_pool_call = pl.pallas_call(
    _pool_body,
    grid=(N // TN,),
    in_specs=[
        pl.BlockSpec((TN, C), lambda j: (j, 0)),
        pl.BlockSpec((TN, C), lambda j: (j, 0)),
        pl.BlockSpec((TN, C), lambda j: (j, 0)),
        pl.BlockSpec((C, C), lambda j: (0, 0)),
        pl.BlockSpec((1, C), lambda j: (0, 0)),
        pl.BlockSpec((1, 1, TN), lambda j: (j, 0, 0)),
    ],
    out_specs=pl.BlockSpec((N_GRAPHS, C), lambda j: (0, 0)),
    out_shape=jax.ShapeDtypeStruct((N_GRAPHS, C), jnp.float32),
)


def _head_body(pooled_ref, lw_ref, lb_ref, w1s_ref, b1s_ref, w2s_ref, b2s_ref,
               sw_ref, sb_ref, rw_ref, rb_ref, o_size_ref, o_rse_ref):
    out = jnp.dot(pooled_ref[...], lw_ref[...], **_DP) + lb_ref[...]
    for li in range(N_LAYERS):
        h = jnp.maximum(jnp.dot(out, w1s_ref[li], **_DP) + b1s_ref[li], 0.0)
        h = jnp.dot(h, w2s_ref[li], **_DP) + b2s_ref[li]
        out = out + h
    o_size_ref[...] = jnp.dot(out, sw_ref[...], **_DP) + sb_ref[...]
    o_rse_ref[...] = jnp.dot(out, rw_ref[...], **_DP) + rb_ref[...]


_head_call = pl.pallas_call(
    _head_body,
    out_shape=(
        jax.ShapeDtypeStruct((N_GRAPHS, 7), jnp.float32),
        jax.ShapeDtypeStruct((N_GRAPHS, 1), jnp.float32),
    ),
)


def kernel(x, edge_index, edge_attr, batch, params):
    f32 = jnp.float32
    src = edge_index[0]
    dst = edge_index[1]
    pad = EP - E
    src_p = jnp.concatenate([src, jnp.zeros((pad,), jnp.int32)]).reshape(NW, K, CH)
    sink = N + (jnp.arange(pad, dtype=jnp.int32) % (NACC - N))
    dst_p = jnp.concatenate([dst, sink]).reshape(NW, K, CH)
    ea_p = jnp.concatenate([edge_attr, jnp.zeros((pad, C), f32)], axis=0)
    zeros_tile = jnp.zeros((RPT, C), f32)
    batch3 = batch.reshape(N // TN, 1, TN)

    sc_gather, sc_scatter = _sc_kernels()

    def conv(xin, p):
        xs = sc_gather(xin, src_p)
        msg = _msg_call(ea_p, xs, p["w1"], p["b1"].reshape(1, HID),
                        p["w2"], p["b2"].reshape(1, H2))
        return sc_scatter(msg, dst_p, zeros_tile)

    p1 = params["conv1"]
    parts1 = conv(x, p1)
    x1 = _combine_call(parts1[:NACC], parts1[NACC:], x,
                       p1["root"], p1["bias"].reshape(1, C))

    p2 = params["conv2"]
    parts2 = conv(x1, p2)
    pooled = _pool_call(parts2[:NACC], parts2[NACC:], x1,
                        p2["root"], p2["bias"].reshape(1, C), batch3)

    w1s = jnp.stack([lp["w1"] for lp in params["layers"]])
    b1s = jnp.stack([lp["b1"].reshape(1, LAYER_DIM) for lp in params["layers"]])
    w2s = jnp.stack([lp["w2"] for lp in params["layers"]])
    b2s = jnp.stack([lp["b2"].reshape(1, LAYER_DIM) for lp in params["layers"]])
    size_logits, rse = _head_call(
        pooled, params["lin0"]["w"], params["lin0"]["b"].reshape(1, LAYER_DIM),
        w1s, b1s, w2s, b2s,
        params["size"]["w"], params["size"]["b"].reshape(1, 7),
        params["rse"]["w"], params["rse"]["b"].reshape(1, 1),
    )
    return (size_logits, rse)


# speedup vs baseline: 3.3709x; 1.1807x over previous
import functools

import jax
import jax.numpy as jnp
from jax import lax
from jax.experimental import pallas as pl
from jax.experimental.pallas import tpu as pltpu
from jax.experimental.pallas import tpu_sc as plsc

N = 10000
E = 160000
C = 16
HID = 64
H2 = C * C
LAYER_DIM = 154
N_LAYERS = 6
N_GRAPHS = 64

NC = 2
NS = 16
NW = NC * NS
CH = 128
K = 40
EPW = K * CH
EP = NW * EPW
RPT = 640
NACC = NS * RPT

TE = 4096
TN = 2000

_DP = dict(preferred_element_type=jnp.float32, precision=lax.Precision.DEFAULT)
_XP = dict(preferred_element_type=jnp.float32, precision=lax.Precision.HIGHEST)


@functools.cache
def _sc_kernels():
    mesh = plsc.VectorSubcoreMesh(core_axis_name="c", subcore_axis_name="s")

    @functools.partial(
        pl.kernel,
        out_type=jax.ShapeDtypeStruct((EP, C), jnp.float32),
        mesh=mesh,
        compiler_params=pltpu.CompilerParams(use_tc_tiling_on_sc=False),
        scratch_types=[
            pltpu.VMEM((K, CH), jnp.int32),
            pltpu.VMEM((EPW, C), jnp.float32),
            pltpu.SemaphoreType.DMA,
        ],
    )
    def sc_gather(table_hbm, idx_hbm, out_hbm, idx_v, rows_buf, sem):
        cid = lax.axis_index("c")
        sid = lax.axis_index("s")
        wid = sid * NC + cid
        pltpu.sync_copy(idx_hbm.at[wid], idx_v)
        base = wid * EPW

        def fire(j, carry):
            pltpu.async_copy(table_hbm.at[idx_v.at[j]],
                             rows_buf.at[pl.ds(j * CH, CH)], sem)
            return carry

        lax.fori_loop(0, K, fire, 0)
        pltpu.make_async_copy(out_hbm.at[pl.ds(base, EPW)], rows_buf, sem).wait()
        pltpu.sync_copy(rows_buf, out_hbm.at[pl.ds(base, EPW)])

    @functools.partial(
        pl.kernel,
        out_type=jax.ShapeDtypeStruct((NC * NACC, C), jnp.float32),
        mesh=mesh,
        compiler_params=pltpu.CompilerParams(use_tc_tiling_on_sc=False),
        scratch_types=[
            pltpu.VMEM((K, CH), jnp.int32),
            pltpu.VMEM((EPW, C), jnp.float32),
            pltpu.VMEM_SHARED((NACC, C), jnp.float32),
            pltpu.SemaphoreType.DMA,
            pltpu.SemaphoreType.DMA,
        ],
    )
    def sc_scatter(msg_hbm, dst_hbm, zeros_hbm, out_hbm,
                   idx_v, msg_buf, acc_sh, sem, sem2):
        cid = lax.axis_index("c")
        sid = lax.axis_index("s")
        wid = sid * NC + cid
        pltpu.sync_copy(zeros_hbm, acc_sh.at[pl.ds(sid * RPT, RPT)])
        pltpu.sync_copy(dst_hbm.at[wid], idx_v)
        base = wid * EPW
        pltpu.sync_copy(msg_hbm.at[pl.ds(base, EPW)], msg_buf)
        plsc.subcore_barrier()

        def fire(j, carry):
            pltpu.async_copy(msg_buf.at[pl.ds(j * CH, CH)],
                             acc_sh.at[idx_v.at[j]], sem, add=True)
            return carry

        lax.fori_loop(0, K, fire, 0)
        pltpu.make_async_copy(msg_hbm.at[pl.ds(base, EPW)], msg_buf, sem).wait()
        plsc.subcore_barrier()
        pltpu.sync_copy(
            acc_sh.at[pl.ds(sid * RPT, RPT)],
            out_hbm.at[pl.ds(cid * NACC + sid * RPT, RPT)],
        )

    return sc_gather, sc_scatter


def _msg_body(ea_ref, xs_ref, w1_ref, b1_ref, w2_ref, b2_ref, o_ref):
    h1 = jnp.maximum(jnp.dot(ea_ref[...], w1_ref[...], **_DP) + b1_ref[...], 0.0)
    hw = jnp.dot(h1, w2_ref[...], **_DP) + b2_ref[...]
    row = lax.broadcasted_iota(jnp.int32, (C, H2), 0)
    col = lax.broadcasted_iota(jnp.int32, (C, H2), 1)
    sel_r = jnp.where(col // C == row, 1.0, 0.0).astype(jnp.float32)
    xe = jnp.dot(xs_ref[...], sel_r, **_DP)
    prod = xe * hw
    cc = lax.broadcasted_iota(jnp.int32, (H2, C), 0)
    oo = lax.broadcasted_iota(jnp.int32, (H2, C), 1)
    sel_s = jnp.where(cc % C == oo, 1.0, 0.0).astype(jnp.float32)
    o_ref[...] = jnp.dot(prod, sel_s, **_DP)


_msg_call = pl.pallas_call(
    _msg_body,
    grid=(EP // TE,),
    in_specs=[
        pl.BlockSpec((TE, C), lambda j: (j, 0)),
        pl.BlockSpec((TE, C), lambda j: (j, 0)),
        pl.BlockSpec((C, HID), lambda j: (0, 0)),
        pl.BlockSpec((1, HID), lambda j: (0, 0)),
        pl.BlockSpec((HID, H2), lambda j: (0, 0)),
        pl.BlockSpec((1, H2), lambda j: (0, 0)),
    ],
    out_specs=pl.BlockSpec((TE, C), lambda j: (j, 0)),
    out_shape=jax.ShapeDtypeStruct((EP, C), jnp.float32),
)


def _combine_body(p0_ref, p1_ref, x_ref, root_ref, bias_ref, o_ref):
    o_ref[...] = (p0_ref[...] + p1_ref[...]
                  + jnp.dot(x_ref[...], root_ref[...], **_DP) + bias_ref[...])


_combine_call = pl.pallas_call(
    _combine_body,
    grid=(N // TN,),
    in_specs=[
        pl.BlockSpec((TN, C), lambda j: (j, 0)),
        pl.BlockSpec((TN, C), lambda j: (j, 0)),
        pl.BlockSpec((TN, C), lambda j: (j, 0)),
        pl.BlockSpec((C, C), lambda j: (0, 0)),
        pl.BlockSpec((1, C), lambda j: (0, 0)),
    ],
    out_specs=pl.BlockSpec((TN, C), lambda j: (j, 0)),
    out_shape=jax.ShapeDtypeStruct((N, C), jnp.float32),
)


def _pool_body(p0_ref, p1_ref, x_ref, root_ref, bias_ref, batch_ref, o_ref):
    j = pl.program_id(0)
    xx = (p0_ref[...] + p1_ref[...]
          + jnp.dot(x_ref[...], root_ref[...], **_DP) + bias_ref[...])
    b = batch_ref[0]
    g = lax.broadcasted_iota(jnp.int32, (N_GRAPHS, TN), 0)
    oh = jnp.where(g == b, 1.0, 0.0).astype(jnp.float32)
    part = jnp.dot(oh, xx, **_XP)

    @pl.when(j == 0)
    def _():
        o_ref[...] = part

    @pl.when(j != 0)
    def _():
        o_ref[...] = o_ref[...] + part


_pool_call = pl.pallas_call(
    _pool_body,
    grid=(N // TN,),
    in_specs=[
        pl.BlockSpec((TN, C), lambda j: (j, 0)),
        pl.BlockSpec((TN, C), lambda j: (j, 0)),
        pl.BlockSpec((TN, C), lambda j: (j, 0)),
        pl.BlockSpec((C, C), lambda j: (0, 0)),
        pl.BlockSpec((1, C), lambda j: (0, 0)),
        pl.BlockSpec((1, 1, TN), lambda j: (j, 0, 0)),
    ],
    out_specs=pl.BlockSpec((N_GRAPHS, C), lambda j: (0, 0)),
    out_shape=jax.ShapeDtypeStruct((N_GRAPHS, C), jnp.float32),
)


def _head_body(pooled_ref, lw_ref, lb_ref, w1s_ref, b1s_ref, w2s_ref, b2s_ref,
               sw_ref, sb_ref, rw_ref, rb_ref, o_size_ref, o_rse_ref):
    out = jnp.dot(pooled_ref[...], lw_ref[...], **_DP) + lb_ref[...]
    for li in range(N_LAYERS):
        h = jnp.maximum(jnp.dot(out, w1s_ref[li], **_DP) + b1s_ref[li], 0.0)
        h = jnp.dot(h, w2s_ref[li], **_DP) + b2s_ref[li]
        out = out + h
    o_size_ref[...] = jnp.dot(out, sw_ref[...], **_DP) + sb_ref[...]
    o_rse_ref[...] = jnp.dot(out, rw_ref[...], **_DP) + rb_ref[...]


_head_call = pl.pallas_call(
    _head_body,
    out_shape=(
        jax.ShapeDtypeStruct((N_GRAPHS, 7), jnp.float32),
        jax.ShapeDtypeStruct((N_GRAPHS, 1), jnp.float32),
    ),
)


def kernel(x, edge_index, edge_attr, batch, params):
    f32 = jnp.float32
    src = edge_index[0]
    dst = edge_index[1]
    pad = EP - E
    src_p = jnp.concatenate([src, jnp.zeros((pad,), jnp.int32)]).reshape(NW, K, CH)
    sink = N + (jnp.arange(pad, dtype=jnp.int32) % (NACC - N))
    dst_p = jnp.concatenate([dst, sink]).reshape(NW, K, CH)
    ea_p = jnp.concatenate([edge_attr, jnp.zeros((pad, C), f32)], axis=0)
    zeros_tile = jnp.zeros((RPT, C), f32)
    batch3 = batch.reshape(N // TN, 1, TN)

    sc_gather, sc_scatter = _sc_kernels()

    def conv(xin, p):
        xs = sc_gather(xin, src_p)
        msg = _msg_call(ea_p, xs, p["w1"], p["b1"].reshape(1, HID),
                        p["w2"], p["b2"].reshape(1, H2))
        return sc_scatter(msg, dst_p, zeros_tile)

    p1 = params["conv1"]
    parts1 = conv(x, p1)
    x1 = _combine_call(parts1[:NACC], parts1[NACC:], x,
                       p1["root"], p1["bias"].reshape(1, C))

    p2 = params["conv2"]
    parts2 = conv(x1, p2)
    pooled = _pool_call(parts2[:NACC], parts2[NACC:], x1,
                        p2["root"], p2["bias"].reshape(1, C), batch3)

    w1s = jnp.stack([lp["w1"] for lp in params["layers"]])
    b1s = jnp.stack([lp["b1"].reshape(1, LAYER_DIM) for lp in params["layers"]])
    w2s = jnp.stack([lp["w2"] for lp in params["layers"]])
    b2s = jnp.stack([lp["b2"].reshape(1, LAYER_DIM) for lp in params["layers"]])
    size_logits, rse = _head_call(
        pooled, params["lin0"]["w"], params["lin0"]["b"].reshape(1, LAYER_DIM),
        w1s, b1s, w2s, b2s,
        params["size"]["w"], params["size"]["b"].reshape(1, 7),
        params["rse"]["w"], params["rse"]["b"].reshape(1, 1),
    )
    return (size_logits, rse)
